# baseline (speedup 1.0000x reference)
import functools

import jax
import jax.numpy as jnp
from jax import lax
from jax.experimental import pallas as pl
from jax.experimental.pallas import tpu as pltpu
from jax.experimental.pallas import tpu_sc as plsc

HID = 128
B = 1024
S = 200
SPAD = 208
NA = 112
NB = 96
LANES = 16

_NC = 2
_NS = 16
NW = _NC * _NS
ROWS_PER_W = B // NW


def _sc_gather_sum(ids_flat, word, pos):
    mesh = plsc.VectorSubcoreMesh(core_axis_name="c", subcore_axis_name="s")
    TILE_TOK = ROWS_PER_W * S

    @functools.partial(
        pl.kernel,
        out_type=jax.ShapeDtypeStruct((B * S, HID), jnp.float32),
        mesh=mesh,
        scratch_types=[
            pltpu.VMEM((TILE_TOK + LANES,), jnp.int32),
            pltpu.VMEM((NA,), jnp.int32),
            pltpu.VMEM((NB,), jnp.int32),
            pltpu.VMEM((NA,), jnp.int32),
            pltpu.VMEM((NB,), jnp.int32),
            pltpu.VMEM((NA,), jnp.int32),
            pltpu.VMEM((NB,), jnp.int32),
            pltpu.VMEM((NA,), jnp.int32),
            pltpu.VMEM((NB,), jnp.int32),
            pltpu.VMEM((SPAD, HID), jnp.float32),
            pltpu.VMEM((SPAD, HID), jnp.float32),
            pltpu.VMEM((SPAD, HID), jnp.float32),
            pltpu.VMEM((SPAD, HID), jnp.float32),
            pltpu.SemaphoreType.DMA,
            pltpu.SemaphoreType.DMA,
        ],
        compiler_params=pltpu.CompilerParams(needs_layout_passes=False),
    )
    def k(ids_hbm, word_hbm, pos_hbm, out_hbm, bigids,
          ida0, idb0, pida0, pidb0, ida1, idb1, pida1, pidb1,
          wbuf0, pbuf0, wbuf1, pbuf1, gsem0, gsem1):
        wid = lax.axis_index("s") * _NC + lax.axis_index("c")
        tbase = wid * TILE_TOK
        pltpu.sync_copy(ids_hbm.at[pl.ds(tbase, TILE_TOK)],
                        bigids.at[pl.ds(0, TILE_TOK)])

        lane = lax.iota(jnp.int32, LANES)
        ntail = S - (SPAD - LANES)
        tailmask = lax.shift_right_logical(
            (ntail - 1) - lane + 16, jnp.int32(4)
        ) & 1

        bufs = ((ida0, idb0, pida0, pidb0, wbuf0, pbuf0, gsem0),
                (ida1, idb1, pida1, pidb1, wbuf1, pbuf1, gsem1))

        def prep(i, p):
            ida, idb, pida, pidb, wbuf, pbuf, gsem = bufs[p]
            ibase = i * S
            carry = jnp.int32(0)
            for c in range(SPAD // LANES):
                v = bigids[pl.ds(ibase + c * LANES, LANES)]
                if c == SPAD // LANES - 1:
                    v = v * tailmask
                m = jnp.minimum(jnp.abs(v), 1)
                cs = plsc.cumsum(m)
                pid = (cs + carry) * m
                carry = carry + jnp.sum(m)
                if c < NA // LANES:
                    ida[pl.ds(c * LANES, LANES)] = v
                    pida[pl.ds(c * LANES, LANES)] = pid
                else:
                    idb[pl.ds(c * LANES - NA, LANES)] = v
                    pidb[pl.ds(c * LANES - NA, LANES)] = pid
            pltpu.async_copy(word_hbm.at[ida], wbuf.at[pl.ds(0, NA)], gsem)
            pltpu.async_copy(word_hbm.at[idb], wbuf.at[pl.ds(NA, NB)], gsem)
            pltpu.async_copy(pos_hbm.at[pida], pbuf.at[pl.ds(0, NA)], gsem)
            pltpu.async_copy(pos_hbm.at[pidb], pbuf.at[pl.ds(NA, NB)], gsem)

        def wait_gathers(p):
            ida, idb, pida, pidb, wbuf, pbuf, gsem = bufs[p]
            pltpu.make_async_copy(word_hbm.at[ida], wbuf.at[pl.ds(0, NA)], gsem).wait()
            pltpu.make_async_copy(word_hbm.at[idb], wbuf.at[pl.ds(NA, NB)], gsem).wait()
            pltpu.make_async_copy(pos_hbm.at[pida], pbuf.at[pl.ds(0, NA)], gsem).wait()
            pltpu.make_async_copy(pos_hbm.at[pidb], pbuf.at[pl.ds(NA, NB)], gsem).wait()

        def consume(i, p):
            ida, idb, pida, pidb, wbuf, pbuf, gsem = bufs[p]
            wait_gathers(p)

            def add_body(t, c2):
                for j in range(HID // LANES):
                    wbuf[t, pl.ds(j * LANES, LANES)] = (
                        wbuf[t, pl.ds(j * LANES, LANES)]
                        + pbuf[t, pl.ds(j * LANES, LANES)])
                return c2
            lax.fori_loop(0, S, add_body, 0)
            pltpu.sync_copy(wbuf.at[pl.ds(0, S)],
                            out_hbm.at[pl.ds(tbase + i * S, S)])

        prep(0, 0)
        prep(1, 1)

        def pair_body(h, c0):
            e = 2 * h
            consume(e, 0)
            prep((e + 2) & (ROWS_PER_W - 1), 0)
            consume(e + 1, 1)
            prep((e + 3) & (ROWS_PER_W - 1), 1)
            return c0

        lax.fori_loop(0, ROWS_PER_W // 2, pair_body, 0)
        wait_gathers(0)
        wait_gathers(1)

    return k(ids_flat, word, pos)


def _tc_layernorm(x, typ0, gamma, beta):
    ROWS = 2048
    n_blocks = (B * S) // ROWS

    def body(x_ref, t_ref, g_ref, b_ref, o_ref):
        x = x_ref[...] + t_ref[...]
        mu = jnp.mean(x, axis=-1, keepdims=True)
        xc = x - mu
        var = jnp.mean(xc * xc, axis=-1, keepdims=True)
        o_ref[...] = xc * lax.rsqrt(var + 1e-12) * g_ref[...] + b_ref[...]

    return pl.pallas_call(
        body,
        grid=(n_blocks,),
        in_specs=[
            pl.BlockSpec((ROWS, HID), lambda i: (i, 0)),
            pl.BlockSpec((1, HID), lambda i: (0, 0)),
            pl.BlockSpec((1, HID), lambda i: (0, 0)),
            pl.BlockSpec((1, HID), lambda i: (0, 0)),
        ],
        out_specs=pl.BlockSpec((ROWS, HID), lambda i: (i, 0)),
        out_shape=jax.ShapeDtypeStruct((B * S, HID), jnp.float32),
    )(x, typ0, gamma, beta)


def kernel(input_ids, word_embeddings, position_embeddings,
           token_type_embeddings, ln_weight, ln_bias):
    ids_flat = input_ids.astype(jnp.int32).reshape(B * S)
    sums = _sc_gather_sum(ids_flat, word_embeddings, position_embeddings)
    typ0 = token_type_embeddings[0:1]
    out = _tc_layernorm(sums, typ0,
                        ln_weight.reshape(1, HID), ln_bias.reshape(1, HID))
    return out.reshape(B, S, HID)


# speedup vs baseline: 2.5364x; 2.5364x over previous
import functools

import jax
import jax.numpy as jnp
from jax import lax
from jax.experimental import pallas as pl
from jax.experimental.pallas import tpu as pltpu
from jax.experimental.pallas import tpu_sc as plsc

HID = 128
B = 1024
S = 200
MAXPOS = 256
NA = 104
NB = 96

_NC = 2
_NS = 16
NW = _NC * _NS
ROWS_PER_W = B // NW


def _sc_gather(ids_flat, word):
    mesh = plsc.VectorSubcoreMesh(core_axis_name="c", subcore_axis_name="s")
    TILE_TOK = ROWS_PER_W * S

    @functools.partial(
        pl.kernel,
        out_type=jax.ShapeDtypeStruct((B * S, HID), jnp.float32),
        mesh=mesh,
        scratch_types=[
            pltpu.VMEM((TILE_TOK,), jnp.int32),
            pltpu.VMEM((S, HID), jnp.float32),
            pltpu.VMEM((S, HID), jnp.float32),
            pltpu.SemaphoreType.DMA,
            pltpu.SemaphoreType.DMA,
            pltpu.SemaphoreType.DMA,
            pltpu.SemaphoreType.DMA,
        ],
        compiler_params=pltpu.CompilerParams(needs_layout_passes=False),
    )
    def k(ids_hbm, word_hbm, out_hbm, bigids, wbuf0, wbuf1,
          gsem0, gsem1, wsem0, wsem1):
        wid = lax.axis_index("s") * _NC + lax.axis_index("c")
        tbase = wid * TILE_TOK
        pltpu.sync_copy(ids_hbm.at[pl.ds(tbase, TILE_TOK)], bigids)

        wbufs = (wbuf0, wbuf1)
        gsems = (gsem0, gsem1)
        wsems = (wsem0, wsem1)

        def fire(i, p):
            ib = i * S
            pltpu.async_copy(word_hbm.at[bigids.at[pl.ds(ib, NA)]],
                             wbufs[p].at[pl.ds(0, NA)], gsems[p])
            pltpu.async_copy(word_hbm.at[bigids.at[pl.ds(ib + NA, NB)]],
                             wbufs[p].at[pl.ds(NA, NB)], gsems[p])

        def wait_gather(i, p):
            ib = i * S
            pltpu.make_async_copy(word_hbm.at[bigids.at[pl.ds(ib, NA)]],
                                  wbufs[p].at[pl.ds(0, NA)], gsems[p]).wait()
            pltpu.make_async_copy(word_hbm.at[bigids.at[pl.ds(ib + NA, NB)]],
                                  wbufs[p].at[pl.ds(NA, NB)], gsems[p]).wait()

        def fire_wb(i, p):
            pltpu.async_copy(wbufs[p], out_hbm.at[pl.ds(tbase + i * S, S)],
                             wsems[p])

        def wait_wb(i, p):
            pltpu.make_async_copy(wbufs[p],
                                  out_hbm.at[pl.ds(tbase + i * S, S)],
                                  wsems[p]).wait()

        fire(0, 0)
        fire(1, 1)

        def pair_body(h, c0):
            e = 2 * h
            wait_gather(e, 0)
            fire_wb(e, 0)
            wait_gather(e + 1, 1)
            fire_wb(e + 1, 1)
            wait_wb(e, 0)
            fire(e + 2, 0)
            wait_wb(e + 1, 1)
            fire(e + 3, 1)
            return c0

        lax.fori_loop(0, ROWS_PER_W // 2 - 1, pair_body, 0)
        last = ROWS_PER_W - 2
        wait_gather(last, 0)
        fire_wb(last, 0)
        wait_gather(last + 1, 1)
        fire_wb(last + 1, 1)
        wait_wb(last, 0)
        wait_wb(last + 1, 1)

    return k(ids_flat, word)


def _tc_posid(ids):
    BBLK = 128
    n_blocks = B // BBLK

    def body(ids_ref, o_ref):
        mask = jnp.where(ids_ref[...] != 0, 1.0, 0.0).astype(jnp.float32)
        r = lax.broadcasted_iota(jnp.int32, (S, S), 0)
        c = lax.broadcasted_iota(jnp.int32, (S, S), 1)
        tri = jnp.where(r <= c, 1.0, 0.0).astype(jnp.float32)
        inc = jax.lax.dot_general(
            mask, tri, (((1,), (0,)), ((), ())),
            preferred_element_type=jnp.float32)
        o_ref[...] = inc * mask

    return pl.pallas_call(
        body,
        grid=(n_blocks,),
        in_specs=[pl.BlockSpec((BBLK, S), lambda i: (i, 0))],
        out_specs=pl.BlockSpec((BBLK, S), lambda i: (i, 0)),
        out_shape=jax.ShapeDtypeStruct((B, S), jnp.float32),
    )(ids)


def _tc_posln(word_rows, posid, pos, typ0, gamma, beta):
    ROWS = 2048
    n_blocks = (B * S) // ROWS

    def body(x_ref, pid_ref, pos_ref, t_ref, g_ref, b_ref, o_ref):
        pid = pid_ref[...]
        pcols = lax.broadcasted_iota(jnp.int32, (1, MAXPOS), 1).astype(jnp.float32)
        onehot = jnp.where(pid == pcols, 1.0, 0.0).astype(jnp.bfloat16)
        pos_emb = jax.lax.dot_general(
            onehot, pos_ref[...].astype(jnp.bfloat16),
            (((1,), (0,)), ((), ())),
            preferred_element_type=jnp.float32)

        x = x_ref[...] + pos_emb + t_ref[...]
        mu = jnp.mean(x, axis=-1, keepdims=True)
        xc = x - mu
        var = jnp.mean(xc * xc, axis=-1, keepdims=True)
        o_ref[...] = xc * lax.rsqrt(var + 1e-12) * g_ref[...] + b_ref[...]

    return pl.pallas_call(
        body,
        grid=(n_blocks,),
        in_specs=[
            pl.BlockSpec((ROWS, HID), lambda i: (i, 0)),
            pl.BlockSpec((ROWS, 1), lambda i: (i, 0)),
            pl.BlockSpec((MAXPOS, HID), lambda i: (0, 0)),
            pl.BlockSpec((1, HID), lambda i: (0, 0)),
            pl.BlockSpec((1, HID), lambda i: (0, 0)),
            pl.BlockSpec((1, HID), lambda i: (0, 0)),
        ],
        out_specs=pl.BlockSpec((ROWS, HID), lambda i: (i, 0)),
        out_shape=jax.ShapeDtypeStruct((B * S, HID), jnp.float32),
    )(word_rows, posid, pos, typ0, gamma, beta)


def kernel(input_ids, word_embeddings, position_embeddings,
           token_type_embeddings, ln_weight, ln_bias):
    ids = input_ids.astype(jnp.int32)
    word_rows = _sc_gather(ids.reshape(B * S), word_embeddings)
    posid = _tc_posid(ids).reshape(B * S, 1)
    typ0 = token_type_embeddings[0:1]
    out = _tc_posln(word_rows, posid, position_embeddings, typ0,
                    ln_weight.reshape(1, HID), ln_bias.reshape(1, HID))
    return out.reshape(B, S, HID)


# speedup vs baseline: 2.7118x; 1.0692x over previous
import functools

import jax
import jax.numpy as jnp
from jax import lax
from jax.experimental import pallas as pl
from jax.experimental.pallas import tpu as pltpu
from jax.experimental.pallas import tpu_sc as plsc

HID = 128
B = 1024
S = 200
SPAD = 208
MAXPOS = 256
NA = 104
NB = 96
LANES = 16

_NC = 2
_NS = 16
NW = _NC * _NS
ROWS_PER_W = B // NW


def _sc_gather(ids_flat, word):
    mesh = plsc.VectorSubcoreMesh(core_axis_name="c", subcore_axis_name="s")
    TILE_TOK = ROWS_PER_W * S

    @functools.partial(
        pl.kernel,
        out_type=(jax.ShapeDtypeStruct((B * S, HID), jnp.float32),
                  jax.ShapeDtypeStruct((B * S,), jnp.int32)),
        mesh=mesh,
        scratch_types=[
            pltpu.VMEM((TILE_TOK + LANES,), jnp.int32),
            pltpu.VMEM((S, HID), jnp.float32),
            pltpu.VMEM((S, HID), jnp.float32),
            pltpu.VMEM((SPAD,), jnp.int32),
            pltpu.VMEM((SPAD,), jnp.int32),
            pltpu.SemaphoreType.DMA,
            pltpu.SemaphoreType.DMA,
            pltpu.SemaphoreType.DMA,
            pltpu.SemaphoreType.DMA,
        ],
        compiler_params=pltpu.CompilerParams(needs_layout_passes=False),
    )
    def k(ids_hbm, word_hbm, out_hbm, pid_hbm, bigids, wbuf0, wbuf1,
          pbuf0, pbuf1, gsem0, gsem1, wsem0, wsem1):
        wid = lax.axis_index("s") * _NC + lax.axis_index("c")
        tbase = wid * TILE_TOK
        pltpu.sync_copy(ids_hbm.at[pl.ds(tbase, TILE_TOK)],
                        bigids.at[pl.ds(0, TILE_TOK)])

        wbufs = (wbuf0, wbuf1)
        pbufs = (pbuf0, pbuf1)
        gsems = (gsem0, gsem1)
        wsems = (wsem0, wsem1)

        lane = lax.iota(jnp.int32, LANES)
        ntail = S - (SPAD - LANES)
        tailmask = lax.shift_right_logical(
            (ntail - 1) - lane + 16, jnp.int32(4)
        ) & 1

        def fire(i, p):
            ib = i * S
            pltpu.async_copy(word_hbm.at[bigids.at[pl.ds(ib, NA)]],
                             wbufs[p].at[pl.ds(0, NA)], gsems[p])
            pltpu.async_copy(word_hbm.at[bigids.at[pl.ds(ib + NA, NB)]],
                             wbufs[p].at[pl.ds(NA, NB)], gsems[p])
            carry = jnp.int32(0)
            for c in range(SPAD // LANES):
                v = bigids[pl.ds(ib + c * LANES, LANES)]
                if c == SPAD // LANES - 1:
                    v = v * tailmask
                m = jnp.minimum(jnp.abs(v), 1)
                cs = plsc.cumsum(m)
                pbufs[p][pl.ds(c * LANES, LANES)] = (cs + carry) * m
                carry = carry + jnp.sum(m)

        def wait_gather(i, p):
            ib = i * S
            pltpu.make_async_copy(word_hbm.at[bigids.at[pl.ds(ib, NA)]],
                                  wbufs[p].at[pl.ds(0, NA)], gsems[p]).wait()
            pltpu.make_async_copy(word_hbm.at[bigids.at[pl.ds(ib + NA, NB)]],
                                  wbufs[p].at[pl.ds(NA, NB)], gsems[p]).wait()

        def fire_wb(i, p):
            pltpu.async_copy(wbufs[p], out_hbm.at[pl.ds(tbase + i * S, S)],
                             wsems[p])
            pltpu.async_copy(pbufs[p].at[pl.ds(0, S)],
                             pid_hbm.at[pl.ds(tbase + i * S, S)], wsems[p])

        def wait_wb(i, p):
            pltpu.make_async_copy(wbufs[p],
                                  out_hbm.at[pl.ds(tbase + i * S, S)],
                                  wsems[p]).wait()
            pltpu.make_async_copy(pbufs[p].at[pl.ds(0, S)],
                                  pid_hbm.at[pl.ds(tbase + i * S, S)],
                                  wsems[p]).wait()

        fire(0, 0)
        fire(1, 1)

        def pair_body(h, c0):
            e = 2 * h
            wait_gather(e, 0)
            fire_wb(e, 0)
            wait_gather(e + 1, 1)
            fire_wb(e + 1, 1)
            wait_wb(e, 0)
            fire(e + 2, 0)
            wait_wb(e + 1, 1)
            fire(e + 3, 1)
            return c0

        lax.fori_loop(0, ROWS_PER_W // 2 - 1, pair_body, 0)
        last = ROWS_PER_W - 2
        wait_gather(last, 0)
        fire_wb(last, 0)
        wait_gather(last + 1, 1)
        fire_wb(last + 1, 1)
        wait_wb(last, 0)
        wait_wb(last + 1, 1)

    return k(ids_flat, word)


def _tc_posln(word_rows, posid, pos, typ0, gamma, beta):
    ROWS = 4096
    n_blocks = (B * S) // ROWS

    def body(x_ref, pid_ref, pos_ref, t_ref, g_ref, b_ref, o_ref):
        pid = pid_ref[...]
        pcols = lax.broadcasted_iota(jnp.int32, (1, MAXPOS), 1)
        onehot = jnp.where(pid == pcols, 1.0, 0.0).astype(jnp.bfloat16)
        pos_emb = jax.lax.dot_general(
            onehot, pos_ref[...].astype(jnp.bfloat16),
            (((1,), (0,)), ((), ())),
            preferred_element_type=jnp.float32)

        x = x_ref[...] + pos_emb + t_ref[...]
        mu = jnp.mean(x, axis=-1, keepdims=True)
        xc = x - mu
        var = jnp.mean(xc * xc, axis=-1, keepdims=True)
        o_ref[...] = xc * lax.rsqrt(var + 1e-12) * g_ref[...] + b_ref[...]

    return pl.pallas_call(
        body,
        grid=(n_blocks,),
        in_specs=[
            pl.BlockSpec((ROWS, HID), lambda i: (i, 0)),
            pl.BlockSpec((ROWS, 1), lambda i: (i, 0)),
            pl.BlockSpec((MAXPOS, HID), lambda i: (0, 0)),
            pl.BlockSpec((1, HID), lambda i: (0, 0)),
            pl.BlockSpec((1, HID), lambda i: (0, 0)),
            pl.BlockSpec((1, HID), lambda i: (0, 0)),
        ],
        out_specs=pl.BlockSpec((ROWS, HID), lambda i: (i, 0)),
        out_shape=jax.ShapeDtypeStruct((B * S, HID), jnp.float32),
    )(word_rows, posid, pos, typ0, gamma, beta)


def kernel(input_ids, word_embeddings, position_embeddings,
           token_type_embeddings, ln_weight, ln_bias):
    ids = input_ids.astype(jnp.int32)
    word_rows, posid = _sc_gather(ids.reshape(B * S), word_embeddings)
    typ0 = token_type_embeddings[0:1]
    out = _tc_posln(word_rows, posid.reshape(B * S, 1), position_embeddings,
                    typ0, ln_weight.reshape(1, HID), ln_bias.reshape(1, HID))
    return out.reshape(B, S, HID)


# speedup vs baseline: 3.2412x; 1.1952x over previous
import functools

import jax
import jax.numpy as jnp
from jax import lax
from jax.experimental import pallas as pl
from jax.experimental.pallas import tpu as pltpu
from jax.experimental.pallas import tpu_sc as plsc

HID = 128
B = 1024
S = 200
SPAD = 208
MAXPOS = 256
NA = 104
NB = 96
LANES = 16

_NC = 2
_NS = 16
NW = _NC * _NS
ROWS_PER_W = B // NW


def _sc_gather(ids_flat, word):
    mesh = plsc.VectorSubcoreMesh(core_axis_name="c", subcore_axis_name="s")
    TILE_TOK = ROWS_PER_W * S

    @functools.partial(
        pl.kernel,
        out_type=(jax.ShapeDtypeStruct((B * S, HID), jnp.float32),
                  jax.ShapeDtypeStruct((B * S,), jnp.int32)),
        mesh=mesh,
        scratch_types=[
            pltpu.VMEM((TILE_TOK + LANES,), jnp.int32),
            pltpu.VMEM((S, HID), jnp.float32),
            pltpu.VMEM((S, HID), jnp.float32),
            pltpu.VMEM((SPAD,), jnp.int32),
            pltpu.VMEM((SPAD,), jnp.int32),
            pltpu.SemaphoreType.DMA,
            pltpu.SemaphoreType.DMA,
            pltpu.SemaphoreType.DMA,
            pltpu.SemaphoreType.DMA,
        ],
        compiler_params=pltpu.CompilerParams(needs_layout_passes=False),
    )
    def k(ids_hbm, word_hbm, out_hbm, pid_hbm, bigids, wbuf0, wbuf1,
          pbuf0, pbuf1, gsem0, gsem1, wsem0, wsem1):
        wid = lax.axis_index("s") * _NC + lax.axis_index("c")
        tbase = wid * TILE_TOK
        pltpu.sync_copy(ids_hbm.at[pl.ds(tbase, TILE_TOK)],
                        bigids.at[pl.ds(0, TILE_TOK)])

        wbufs = (wbuf0, wbuf1)
        pbufs = (pbuf0, pbuf1)
        gsems = (gsem0, gsem1)
        wsems = (wsem0, wsem1)

        lane = lax.iota(jnp.int32, LANES)
        ntail = S - (SPAD - LANES)
        tailmask = lax.shift_right_logical(
            (ntail - 1) - lane + 16, jnp.int32(4)
        ) & 1

        def fire(i, p):
            ib = i * S
            pltpu.async_copy(word_hbm.at[bigids.at[pl.ds(ib, NA)]],
                             wbufs[p].at[pl.ds(0, NA)], gsems[p])
            pltpu.async_copy(word_hbm.at[bigids.at[pl.ds(ib + NA, NB)]],
                             wbufs[p].at[pl.ds(NA, NB)], gsems[p])
            carry = jnp.int32(0)
            for c in range(SPAD // LANES):
                v = bigids[pl.ds(ib + c * LANES, LANES)]
                if c == SPAD // LANES - 1:
                    v = v * tailmask
                m = jnp.minimum(jnp.abs(v), 1)
                cs = plsc.cumsum(m)
                pbufs[p][pl.ds(c * LANES, LANES)] = (cs + carry) * m
                carry = carry + jnp.sum(m)

        def wait_gather(i, p):
            ib = i * S
            pltpu.make_async_copy(word_hbm.at[bigids.at[pl.ds(ib, NA)]],
                                  wbufs[p].at[pl.ds(0, NA)], gsems[p]).wait()
            pltpu.make_async_copy(word_hbm.at[bigids.at[pl.ds(ib + NA, NB)]],
                                  wbufs[p].at[pl.ds(NA, NB)], gsems[p]).wait()

        def fire_wb(i, p):
            pltpu.async_copy(wbufs[p], out_hbm.at[pl.ds(tbase + i * S, S)],
                             wsems[p])
            pltpu.async_copy(pbufs[p].at[pl.ds(0, S)],
                             pid_hbm.at[pl.ds(tbase + i * S, S)], wsems[p])

        def wait_wb(i, p):
            pltpu.make_async_copy(wbufs[p],
                                  out_hbm.at[pl.ds(tbase + i * S, S)],
                                  wsems[p]).wait()
            pltpu.make_async_copy(pbufs[p].at[pl.ds(0, S)],
                                  pid_hbm.at[pl.ds(tbase + i * S, S)],
                                  wsems[p]).wait()

        fire(0, 0)
        fire(1, 1)

        def pair_body(h, c0):
            e = 2 * h
            wait_gather(e, 0)
            fire_wb(e, 0)
            wait_gather(e + 1, 1)
            fire_wb(e + 1, 1)
            wait_wb(e, 0)
            fire(e + 2, 0)
            wait_wb(e + 1, 1)
            fire(e + 3, 1)
            return c0

        lax.fori_loop(0, ROWS_PER_W // 2 - 1, pair_body, 0)
        last = ROWS_PER_W - 2
        wait_gather(last, 0)
        fire_wb(last, 0)
        wait_gather(last + 1, 1)
        fire_wb(last + 1, 1)
        wait_wb(last, 0)
        wait_wb(last + 1, 1)

    return k(ids_flat, word)


def _tc_posln(word_rows, posid3, pos_adj, gamma, beta):
    ROWS = 4096
    n_blocks = (B * S) // ROWS
    INV = 1.0 / HID

    def body(x_ref, pid_ref, pos_ref, g_ref, b_ref, o_ref):
        pid = pid_ref[0]
        prow = lax.broadcasted_iota(jnp.int32, (MAXPOS, 1), 0)
        onehot_t = jnp.where(prow == pid, 1.0, 0.0).astype(jnp.bfloat16)
        pos_emb = jax.lax.dot_general(
            onehot_t, pos_ref[...].astype(jnp.bfloat16),
            (((0,), (0,)), ((), ())),
            preferred_element_type=jnp.float32)

        x = x_ref[...] + pos_emb
        c = lax.broadcasted_iota(jnp.int32, (HID, HID), 1)
        w = jnp.where(c == 0, INV, 0.0).astype(jnp.float32)
        mu = jax.lax.dot_general(
            x, w, (((1,), (0,)), ((), ())),
            preferred_element_type=jnp.float32)[:, 0:1]
        m2 = jax.lax.dot_general(
            x * x, w, (((1,), (0,)), ((), ())),
            preferred_element_type=jnp.float32)[:, 0:1]
        var = jnp.maximum(m2 - mu * mu, 0.0)
        o_ref[...] = (x - mu) * lax.rsqrt(var + 1e-12) * g_ref[...] + b_ref[...]

    return pl.pallas_call(
        body,
        grid=(n_blocks,),
        in_specs=[
            pl.BlockSpec((ROWS, HID), lambda i: (i, 0)),
            pl.BlockSpec((1, 1, ROWS), lambda i: (i, 0, 0)),
            pl.BlockSpec((MAXPOS, HID), lambda i: (0, 0)),
            pl.BlockSpec((1, HID), lambda i: (0, 0)),
            pl.BlockSpec((1, HID), lambda i: (0, 0)),
        ],
        out_specs=pl.BlockSpec((ROWS, HID), lambda i: (i, 0)),
        out_shape=jax.ShapeDtypeStruct((B * S, HID), jnp.float32),
    )(word_rows, posid3, pos_adj, gamma, beta)


def kernel(input_ids, word_embeddings, position_embeddings,
           token_type_embeddings, ln_weight, ln_bias):
    ids = input_ids.astype(jnp.int32)
    word_rows, posid = _sc_gather(ids.reshape(B * S), word_embeddings)
    ROWS = 4096
    posid3 = posid.reshape((B * S) // ROWS, 1, ROWS)
    pos_adj = position_embeddings + token_type_embeddings[0:1]
    out = _tc_posln(word_rows, posid3, pos_adj,
                    ln_weight.reshape(1, HID), ln_bias.reshape(1, HID))
    return out.reshape(B, S, HID)


# speedup vs baseline: 3.2487x; 1.0023x over previous
import functools

import jax
import jax.numpy as jnp
from jax import lax
from jax.experimental import pallas as pl
from jax.experimental.pallas import tpu as pltpu
from jax.experimental.pallas import tpu_sc as plsc

HID = 128
B = 1024
S = 200
SPAD = 208
MAXPOS = 256
NA = 104
NB = 96
LANES = 16

_NC = 2
_NS = 16
NW = _NC * _NS
ROWS_PER_W = B // NW


def _sc_gather(ids_flat, word):
    mesh = plsc.VectorSubcoreMesh(core_axis_name="c", subcore_axis_name="s")
    TILE_TOK = ROWS_PER_W * S

    @functools.partial(
        pl.kernel,
        out_type=(jax.ShapeDtypeStruct((B * S, HID), jnp.float32),
                  jax.ShapeDtypeStruct((B * S,), jnp.int32)),
        mesh=mesh,
        scratch_types=[
            pltpu.VMEM((TILE_TOK + LANES,), jnp.int32),
            pltpu.VMEM((S, HID), jnp.float32),
            pltpu.VMEM((S, HID), jnp.float32),
            pltpu.VMEM((SPAD,), jnp.int32),
            pltpu.VMEM((SPAD,), jnp.int32),
            pltpu.SemaphoreType.DMA,
            pltpu.SemaphoreType.DMA,
            pltpu.SemaphoreType.DMA,
            pltpu.SemaphoreType.DMA,
        ],
        compiler_params=pltpu.CompilerParams(needs_layout_passes=False),
    )
    def k(ids_hbm, word_hbm, out_hbm, pid_hbm, bigids, wbuf0, wbuf1,
          pbuf0, pbuf1, gsem0, gsem1, wsem0, wsem1):
        wid = lax.axis_index("s") * _NC + lax.axis_index("c")
        tbase = wid * TILE_TOK
        pltpu.sync_copy(ids_hbm.at[pl.ds(tbase, TILE_TOK)],
                        bigids.at[pl.ds(0, TILE_TOK)])

        wbufs = (wbuf0, wbuf1)
        pbufs = (pbuf0, pbuf1)
        gsems = (gsem0, gsem1)
        wsems = (wsem0, wsem1)

        lane = lax.iota(jnp.int32, LANES)
        ntail = S - (SPAD - LANES)
        tailmask = lax.shift_right_logical(
            (ntail - 1) - lane + 16, jnp.int32(4)
        ) & 1

        def fire(i, p):
            ib = i * S
            pltpu.async_copy(word_hbm.at[bigids.at[pl.ds(ib, NA)]],
                             wbufs[p].at[pl.ds(0, NA)], gsems[p])
            pltpu.async_copy(word_hbm.at[bigids.at[pl.ds(ib + NA, NB)]],
                             wbufs[p].at[pl.ds(NA, NB)], gsems[p])
            carry = jnp.int32(0)
            for c in range(SPAD // LANES):
                v = bigids[pl.ds(ib + c * LANES, LANES)]
                if c == SPAD // LANES - 1:
                    v = v * tailmask
                m = jnp.minimum(jnp.abs(v), 1)
                cs = plsc.cumsum(m)
                pbufs[p][pl.ds(c * LANES, LANES)] = (cs + carry) * m
                carry = carry + jnp.sum(m)

        def wait_gather(i, p):
            ib = i * S
            pltpu.make_async_copy(word_hbm.at[bigids.at[pl.ds(ib, NA)]],
                                  wbufs[p].at[pl.ds(0, NA)], gsems[p]).wait()
            pltpu.make_async_copy(word_hbm.at[bigids.at[pl.ds(ib + NA, NB)]],
                                  wbufs[p].at[pl.ds(NA, NB)], gsems[p]).wait()

        def fire_wb(i, p):
            pltpu.async_copy(wbufs[p], out_hbm.at[pl.ds(tbase + i * S, S)],
                             wsems[p])
            pltpu.async_copy(pbufs[p].at[pl.ds(0, S)],
                             pid_hbm.at[pl.ds(tbase + i * S, S)], wsems[p])

        def wait_wb(i, p):
            pltpu.make_async_copy(wbufs[p],
                                  out_hbm.at[pl.ds(tbase + i * S, S)],
                                  wsems[p]).wait()
            pltpu.make_async_copy(pbufs[p].at[pl.ds(0, S)],
                                  pid_hbm.at[pl.ds(tbase + i * S, S)],
                                  wsems[p]).wait()

        fire(0, 0)
        fire(1, 1)

        def pair_body(h, c0):
            e = 2 * h
            wait_gather(e, 0)
            fire_wb(e, 0)
            wait_gather(e + 1, 1)
            fire_wb(e + 1, 1)
            wait_wb(e, 0)
            fire(e + 2, 0)
            wait_wb(e + 1, 1)
            fire(e + 3, 1)
            return c0

        lax.fori_loop(0, ROWS_PER_W // 2 - 1, pair_body, 0)
        last = ROWS_PER_W - 2
        wait_gather(last, 0)
        fire_wb(last, 0)
        wait_gather(last + 1, 1)
        fire_wb(last + 1, 1)
        wait_wb(last, 0)
        wait_wb(last + 1, 1)

    return k(ids_flat, word)


def _tc_posln(word_rows, posid3, pos_adj, gamma, beta):
    ROWS = 4096
    n_blocks = (B * S) // ROWS
    INV = 1.0 / HID

    def body(x_ref, pid_ref, pos_ref, g_ref, b_ref, o_ref):
        pid = pid_ref[0]
        prow = lax.broadcasted_iota(jnp.int32, (MAXPOS, 1), 0)
        onehot_t = jnp.where(prow == pid, 1.0, 0.0).astype(jnp.bfloat16)
        pos_emb = jax.lax.dot_general(
            onehot_t, pos_ref[...],
            (((0,), (0,)), ((), ())),
            preferred_element_type=jnp.float32)

        x = x_ref[...] + pos_emb
        c = lax.broadcasted_iota(jnp.int32, (HID, HID), 1)
        w = jnp.where(c == 0, INV, 0.0).astype(jnp.bfloat16)
        xb = x.astype(jnp.bfloat16)
        mu = jax.lax.dot_general(
            xb, w, (((1,), (0,)), ((), ())),
            preferred_element_type=jnp.float32)[:, 0:1]
        m2 = jax.lax.dot_general(
            xb * xb, w, (((1,), (0,)), ((), ())),
            preferred_element_type=jnp.float32)[:, 0:1]
        var = jnp.maximum(m2 - mu * mu, 0.0)
        o_ref[...] = (x - mu) * lax.rsqrt(var + 1e-12) * g_ref[...] + b_ref[...]

    return pl.pallas_call(
        body,
        grid=(n_blocks,),
        in_specs=[
            pl.BlockSpec((ROWS, HID), lambda i: (i, 0)),
            pl.BlockSpec((1, 1, ROWS), lambda i: (i, 0, 0)),
            pl.BlockSpec((MAXPOS, HID), lambda i: (0, 0)),
            pl.BlockSpec((1, HID), lambda i: (0, 0)),
            pl.BlockSpec((1, HID), lambda i: (0, 0)),
        ],
        out_specs=pl.BlockSpec((ROWS, HID), lambda i: (i, 0)),
        out_shape=jax.ShapeDtypeStruct((B * S, HID), jnp.float32),
    )(word_rows, posid3, pos_adj, gamma, beta)


def kernel(input_ids, word_embeddings, position_embeddings,
           token_type_embeddings, ln_weight, ln_bias):
    ids = input_ids.astype(jnp.int32)
    word_rows, posid = _sc_gather(ids.reshape(B * S), word_embeddings)
    ROWS = 4096
    posid3 = posid.reshape((B * S) // ROWS, 1, ROWS)
    pos_adj = (position_embeddings
               + token_type_embeddings[0:1]).astype(jnp.bfloat16)
    out = _tc_posln(word_rows, posid3, pos_adj,
                    ln_weight.reshape(1, HID), ln_bias.reshape(1, HID))
    return out.reshape(B, S, HID)
